# baseline (speedup 1.0000x reference)
import jax
import jax.numpy as jnp
from jax.experimental import pallas as pl
from jax.experimental.pallas import tpu as pltpu


def _fused_kernel(x_ref, w_ref, b_ref, adj_ref, out_ref, adj_out_ref, h_scr):
    @pl.when(pl.program_id(0) == 0)
    def _():
        h = jnp.dot(x_ref[...], w_ref[...], preferred_element_type=jnp.float32)
        h_scr[...] = (h + b_ref[...]).astype(jnp.bfloat16)

    a = adj_ref[...]
    adj_out_ref[...] = a
    acc = jnp.dot(a.astype(jnp.bfloat16), h_scr[...],
                  preferred_element_type=jnp.float32)
    out_ref[...] = jnp.maximum(acc, 0.0)


def kernel(x, adj, W, b):
    n, f_in = x.shape
    f_out = W.shape[1]
    bm = 80

    out, adj_out = pl.pallas_call(
        _fused_kernel,
        grid=(n // bm,),
        in_specs=[
            pl.BlockSpec((n, f_in), lambda i: (0, 0)),
            pl.BlockSpec((f_in, f_out), lambda i: (0, 0)),
            pl.BlockSpec((1, f_out), lambda i: (0, 0)),
            pl.BlockSpec((bm, n), lambda i: (i, 0)),
        ],
        out_specs=[
            pl.BlockSpec((bm, f_out), lambda i: (i, 0)),
            pl.BlockSpec((bm, n), lambda i: (i, 0)),
        ],
        out_shape=[
            jax.ShapeDtypeStruct((n, f_out), jnp.float32),
            jax.ShapeDtypeStruct((n, n), jnp.float32),
        ],
        scratch_shapes=[pltpu.VMEM((n, f_out), jnp.bfloat16)],
    )(x, W, b.reshape(1, f_out), adj)

    return (out, adj_out)


# speedup vs baseline: 1.0156x; 1.0156x over previous
import jax
import jax.numpy as jnp
from jax.experimental import pallas as pl
from jax.experimental.pallas import tpu as pltpu


def _fused_kernel(x_ref, w_ref, b_ref, adj_ref, out_ref, adj_out_ref,
                  h_scr, cp_scr, sem):
    i = pl.program_id(0)
    nsteps = pl.num_programs(0)
    bm = adj_ref.shape[0]

    @pl.when(i == 0)
    def _():
        h = jnp.dot(x_ref[...], w_ref[...], preferred_element_type=jnp.float32)
        h_scr[...] = (h + b_ref[...]).astype(jnp.bfloat16)

    half = bm // 2

    @pl.when(i > 0)
    def _():
        pltpu.make_async_copy(
            cp_scr, adj_out_ref.at[pl.ds((i - 1) * bm + half, half), :], sem,
        ).wait()

    a = adj_ref[...]
    cp_scr[...] = a[:half]
    c1 = pltpu.make_async_copy(
        cp_scr, adj_out_ref.at[pl.ds(i * bm, half), :], sem,
    )
    c1.start()

    acc = jnp.dot(a.astype(jnp.bfloat16), h_scr[...],
                  preferred_element_type=jnp.float32)
    out_ref[...] = jnp.maximum(acc, 0.0)

    c1.wait()
    cp_scr[...] = a[half:]
    c2 = pltpu.make_async_copy(
        cp_scr, adj_out_ref.at[pl.ds(i * bm + half, half), :], sem,
    )
    c2.start()

    @pl.when(i == nsteps - 1)
    def _():
        c2.wait()


def kernel(x, adj, W, b):
    n, f_in = x.shape
    f_out = W.shape[1]
    bm = 400

    out, adj_out = pl.pallas_call(
        _fused_kernel,
        grid=(n // bm,),
        in_specs=[
            pl.BlockSpec((n, f_in), lambda i: (0, 0)),
            pl.BlockSpec((f_in, f_out), lambda i: (0, 0)),
            pl.BlockSpec((1, f_out), lambda i: (0, 0)),
            pl.BlockSpec((bm, n), lambda i: (i, 0)),
        ],
        out_specs=[
            pl.BlockSpec((bm, f_out), lambda i: (i, 0)),
            pl.BlockSpec(memory_space=pl.ANY),
        ],
        out_shape=[
            jax.ShapeDtypeStruct((n, f_out), jnp.float32),
            jax.ShapeDtypeStruct((n, n), jnp.float32),
        ],
        scratch_shapes=[
            pltpu.VMEM((n, f_out), jnp.bfloat16),
            pltpu.VMEM((bm // 2, n), jnp.float32),
            pltpu.SemaphoreType.DMA,
        ],
        compiler_params=pltpu.CompilerParams(
            vmem_limit_bytes=64 * 1024 * 1024,
        ),
    )(x, W, b.reshape(1, f_out), adj)

    return (out, adj_out)


# speedup vs baseline: 1.1008x; 1.0839x over previous
import jax
import jax.numpy as jnp
from jax.experimental import pallas as pl
from jax.experimental.pallas import tpu as pltpu


def _fused_kernel(x_ref, w_ref, b_ref, adj_ref, out_ref, adj_out_ref, h_scr):
    @pl.when(pl.program_id(0) == 0)
    def _():
        h = jnp.dot(x_ref[...], w_ref[...], preferred_element_type=jnp.float32)
        h_scr[...] = (h + b_ref[...]).astype(jnp.bfloat16)

    a = adj_ref[...]
    adj_out_ref[...] = a
    acc = jnp.dot(a.astype(jnp.bfloat16), h_scr[...],
                  preferred_element_type=jnp.float32)
    out_ref[...] = jnp.maximum(acc, 0.0)


def kernel(x, adj, W, b):
    n, f_in = x.shape
    f_out = W.shape[1]
    bm = 200

    out, adj_out = pl.pallas_call(
        _fused_kernel,
        grid=(n // bm,),
        in_specs=[
            pl.BlockSpec((n, f_in), lambda i: (0, 0)),
            pl.BlockSpec((f_in, f_out), lambda i: (0, 0)),
            pl.BlockSpec((1, f_out), lambda i: (0, 0)),
            pl.BlockSpec((bm, n), lambda i: (i, 0)),
        ],
        out_specs=[
            pl.BlockSpec((bm, f_out), lambda i: (i, 0)),
            pl.BlockSpec((bm, n), lambda i: (i, 0)),
        ],
        out_shape=[
            jax.ShapeDtypeStruct((n, f_out), jnp.float32),
            jax.ShapeDtypeStruct((n, n), jnp.float32),
        ],
        scratch_shapes=[pltpu.VMEM((n, f_out), jnp.bfloat16)],
    )(x, W, b.reshape(1, f_out), adj)

    return (out, adj_out)
